# Initial kernel scaffold; baseline (speedup 1.0000x reference)
import jax
import jax.numpy as jnp
from jax.experimental import pallas as pl


def kernel(x, thresholds):
    raise NotImplementedError("write your pallas kernel here")


# speedup vs baseline: 91.1794x; 91.1794x over previous
import functools

import jax
import jax.numpy as jnp
from jax import lax
from jax.experimental import pallas as pl
from jax.experimental.pallas import tpu as pltpu
from jax.experimental.pallas import tpu_sc as plsc

_LANES = 16


def _build_consts(thresholds):
    d = jnp.diff(thresholds, axis=1)
    d = jnp.concatenate([-d[:, :1], d, d[:, -1:]], axis=1)
    th_full = jnp.concatenate([thresholds[:, :1], thresholds], axis=1)
    tr = th_full + d / 2.0
    tr0 = tr[:, :1].T
    dtr = jnp.diff(tr, axis=1).T
    th_t = thresholds.T
    return jnp.concatenate([tr0, th_t, dtr], axis=0).reshape(-1)


@functools.partial(jax.jit, static_argnames=("n_th", "rows_per_w", "chunk"))
def _run(x_flat, consts, n_th, rows_per_w, chunk):
    total = x_flat.shape[0]
    T = n_th
    n_chunks = rows_per_w // chunk
    mesh = plsc.VectorSubcoreMesh(core_axis_name="c", subcore_axis_name="s")

    @functools.partial(
        pl.kernel,
        mesh=mesh,
        out_type=jax.ShapeDtypeStruct((total,), jnp.float32),
        scratch_types=[
            pltpu.VMEM(((1 + 2 * T) * _LANES,), jnp.float32),
            pltpu.VMEM((chunk * _LANES,), jnp.float32),
            pltpu.VMEM((chunk * _LANES,), jnp.float32),
        ],
    )
    def run(x_hbm, c_hbm, out_hbm, c_v, xbuf, obuf):
        wid = lax.axis_index("s") * 2 + lax.axis_index("c")
        base = wid * rows_per_w * _LANES
        pltpu.sync_copy(c_hbm, c_v)
        tr0 = c_v[pl.ds(0, _LANES)]
        ths = [c_v[pl.ds((1 + t) * _LANES, _LANES)] for t in range(T)]
        dtrs = [c_v[pl.ds((1 + T + t) * _LANES, _LANES)] for t in range(T)]

        def chunk_body(ci, carry):
            start = base + ci * (chunk * _LANES)
            pltpu.sync_copy(x_hbm.at[pl.ds(start, chunk * _LANES)], xbuf)

            def row_body(r, c2):
                xv = xbuf[pl.ds(r * _LANES, _LANES)]
                acc = tr0
                for t in range(T):
                    acc = jnp.where(xv > ths[t], acc + dtrs[t], acc)
                obuf[pl.ds(r * _LANES, _LANES)] = acc
                return c2

            lax.fori_loop(0, chunk, row_body, 0, unroll=4)
            pltpu.sync_copy(obuf, out_hbm.at[pl.ds(start, chunk * _LANES)])
            return carry

        lax.fori_loop(0, n_chunks, chunk_body, 0)

    return run(x_flat, consts)


def kernel(x, thresholds):
    B, F = x.shape
    T = thresholds.shape[1]
    consts = _build_consts(thresholds)
    info = plsc.get_sparse_core_info()
    n_workers = info.num_cores * info.num_subcores
    rows_per_w = B // n_workers
    chunk = min(1024, rows_per_w)
    out = _run(x.reshape(-1), consts, T, rows_per_w, chunk)
    return out.reshape(B, F)


# speedup vs baseline: 142.1124x; 1.5586x over previous
import functools

import jax
import jax.numpy as jnp
from jax import lax
from jax.experimental import pallas as pl
from jax.experimental.pallas import tpu as pltpu
from jax.experimental.pallas import tpu_sc as plsc

_LANES = 16


def _build_consts(thresholds):
    d = jnp.diff(thresholds, axis=1)
    d = jnp.concatenate([-d[:, :1], d, d[:, -1:]], axis=1)
    th_full = jnp.concatenate([thresholds[:, :1], thresholds], axis=1)
    tr = th_full + d / 2.0
    tr0 = tr[:, :1].T
    dtr = jnp.diff(tr, axis=1).T
    th_t = thresholds.T
    return jnp.concatenate([tr0, th_t, dtr], axis=0).reshape(-1)


@functools.partial(jax.jit, static_argnames=("n_th", "rows_per_w", "chunk"))
def _run(x_flat, consts, n_th, rows_per_w, chunk):
    total = x_flat.shape[0]
    T = n_th
    n_chunks = rows_per_w // chunk
    cn = chunk * _LANES
    mesh = plsc.VectorSubcoreMesh(core_axis_name="c", subcore_axis_name="s")

    @functools.partial(
        pl.kernel,
        mesh=mesh,
        out_type=jax.ShapeDtypeStruct((total,), jnp.float32),
        scratch_types=[
            pltpu.VMEM(((1 + 2 * T) * _LANES,), jnp.float32),
            pltpu.VMEM((cn,), jnp.float32),
            pltpu.VMEM((cn,), jnp.float32),
            pltpu.VMEM((cn,), jnp.float32),
            pltpu.VMEM((cn,), jnp.float32),
            pltpu.SemaphoreType.DMA,
            pltpu.SemaphoreType.DMA,
            pltpu.SemaphoreType.DMA,
            pltpu.SemaphoreType.DMA,
        ],
    )
    def run(x_hbm, c_hbm, out_hbm, c_v, xb0, xb1, ob0, ob1, si0, si1, so0, so1):
        wid = lax.axis_index("s") * 2 + lax.axis_index("c")
        base = wid * rows_per_w * _LANES
        pltpu.sync_copy(c_hbm, c_v)
        tr0 = c_v[pl.ds(0, _LANES)]
        ths = [c_v[pl.ds((1 + t) * _LANES, _LANES)] for t in range(T)]
        dtrs = [c_v[pl.ds((1 + T + t) * _LANES, _LANES)] for t in range(T)]

        xbufs, obufs = (xb0, xb1), (ob0, ob1)
        isems, osems = (si0, si1), (so0, so1)
        in_h = [None] * n_chunks
        out_h = [None] * n_chunks

        in_h[0] = pltpu.async_copy(x_hbm.at[pl.ds(base, cn)], xbufs[0], isems[0])
        for ci in range(n_chunks):
            cur = ci & 1
            if ci + 1 < n_chunks:
                nxt = base + (ci + 1) * cn
                in_h[ci + 1] = pltpu.async_copy(
                    x_hbm.at[pl.ds(nxt, cn)], xbufs[1 - cur], isems[1 - cur])
            in_h[ci].wait()
            if ci >= 2:
                out_h[ci - 2].wait()
            xbuf, obuf = xbufs[cur], obufs[cur]

            @plsc.parallel_loop(0, chunk, 1, unroll=8)
            def row_body(r):
                xv = xbuf[pl.ds(r * _LANES, _LANES)]
                acc = tr0
                for t in range(T):
                    acc = jnp.where(xv > ths[t], acc + dtrs[t], acc)
                obuf[pl.ds(r * _LANES, _LANES)] = acc

            out_h[ci] = pltpu.async_copy(
                obuf, out_hbm.at[pl.ds(base + ci * cn, cn)], osems[cur])
        for ci in (n_chunks - 2, n_chunks - 1):
            if ci >= 0:
                out_h[ci].wait()

    return run(x_flat, consts)


def kernel(x, thresholds):
    B, F = x.shape
    T = thresholds.shape[1]
    consts = _build_consts(thresholds)
    info = plsc.get_sparse_core_info()
    n_workers = info.num_cores * info.num_subcores
    rows_per_w = B // n_workers
    chunk = min(1024, rows_per_w)
    out = _run(x.reshape(-1), consts, T, rows_per_w, chunk)
    return out.reshape(B, F)


# speedup vs baseline: 448.1661x; 3.1536x over previous
import functools

import jax
import jax.numpy as jnp
from jax import lax
from jax.experimental import pallas as pl
from jax.experimental.pallas import tpu as pltpu
from jax.experimental.pallas import tpu_sc as plsc

_LANES = 16


def _build_consts(thresholds):
    F, T = thresholds.shape
    d = jnp.diff(thresholds, axis=1)
    d = jnp.concatenate([-d[:, :1], d, d[:, -1:]], axis=1)
    th_full = jnp.concatenate([thresholds[:, :1], thresholds], axis=1)
    tr = th_full + d / 2.0
    tr0 = tr[:, :1]
    dtr = jnp.diff(tr, axis=1)
    pad = jnp.zeros((F, 1), jnp.float32)
    return jnp.concatenate([tr0, thresholds, dtr, pad], axis=1)


@functools.partial(jax.jit, static_argnames=("n_th", "cols_per_w", "chunk"))
def _run(xt, consts, n_th, cols_per_w, chunk):
    F, B = xt.shape
    T = n_th
    n_chunks = cols_per_w // chunk
    w_per_f = B // cols_per_w
    mesh = plsc.VectorSubcoreMesh(core_axis_name="c", subcore_axis_name="s")

    @functools.partial(
        pl.kernel,
        mesh=mesh,
        out_type=jax.ShapeDtypeStruct((F, B), jnp.float32),
        scratch_types=[
            pltpu.VMEM((2 * T + 2,), jnp.float32),
            pltpu.VMEM((chunk,), jnp.float32),
            pltpu.VMEM((chunk,), jnp.float32),
            pltpu.VMEM((chunk,), jnp.float32),
            pltpu.VMEM((chunk,), jnp.float32),
            pltpu.SemaphoreType.DMA,
            pltpu.SemaphoreType.DMA,
            pltpu.SemaphoreType.DMA,
            pltpu.SemaphoreType.DMA,
        ],
    )
    def run(x_hbm, c_hbm, out_hbm, c_v, xb0, xb1, ob0, ob1, si0, si1, so0, so1):
        wid = lax.axis_index("s") * 2 + lax.axis_index("c")
        f = wid // w_per_f
        col0 = (wid % w_per_f) * cols_per_w
        pltpu.sync_copy(c_hbm.at[f], c_v)
        ca = c_v[pl.ds(0, _LANES)]
        cb = c_v[pl.ds(_LANES, _LANES)]
        tr0 = ca[0]
        ths = [ca[1 + t] for t in range(T)]
        dtrs = [cb[t] for t in range(T)]

        xbufs, obufs = (xb0, xb1), (ob0, ob1)
        isems, osems = (si0, si1), (so0, so1)
        in_h = [None] * n_chunks
        out_h = [None] * n_chunks

        in_h[0] = pltpu.async_copy(
            x_hbm.at[f, pl.ds(col0, chunk)], xbufs[0], isems[0])
        for ci in range(n_chunks):
            cur = ci & 1
            if ci + 1 < n_chunks:
                nxt = col0 + (ci + 1) * chunk
                in_h[ci + 1] = pltpu.async_copy(
                    x_hbm.at[f, pl.ds(nxt, chunk)], xbufs[1 - cur],
                    isems[1 - cur])
            in_h[ci].wait()
            if ci >= 2:
                out_h[ci - 2].wait()
            xbuf, obuf = xbufs[cur], obufs[cur]

            @plsc.parallel_loop(0, chunk // _LANES, 1, unroll=8)
            def vec_body(r):
                xv = xbuf[pl.ds(r * _LANES, _LANES)]
                acc = jnp.zeros((_LANES,), jnp.float32) + tr0
                for t in range(T):
                    acc = jnp.where(xv > ths[t], acc + dtrs[t], acc)
                obuf[pl.ds(r * _LANES, _LANES)] = acc

            out_h[ci] = pltpu.async_copy(
                obuf, out_hbm.at[f, pl.ds(col0 + ci * chunk, chunk)],
                osems[cur])
        for ci in (n_chunks - 2, n_chunks - 1):
            if ci >= 0:
                out_h[ci].wait()

    return run(xt, consts)


def kernel(x, thresholds):
    B, F = x.shape
    T = thresholds.shape[1]
    consts = _build_consts(thresholds)
    info = plsc.get_sparse_core_info()
    n_workers = info.num_cores * info.num_subcores
    cols_per_w = B // (n_workers // F)
    chunk = 8192
    out_t = _run(x.T, consts, T, cols_per_w, chunk)
    return out_t.T


# speedup vs baseline: 853.8807x; 1.9053x over previous
import functools

import jax
import jax.numpy as jnp
from jax import lax
from jax.experimental import pallas as pl
from jax.experimental.pallas import tpu as pltpu
from jax.experimental.pallas import tpu_sc as plsc

_LANES = 16


def _build_consts(thresholds):
    F, T = thresholds.shape
    d = jnp.diff(thresholds, axis=1)
    d = jnp.concatenate([-d[:, :1], d, d[:, -1:]], axis=1)
    th_full = jnp.concatenate([thresholds[:, :1], thresholds], axis=1)
    tr = th_full + d / 2.0
    pad = jnp.full((F, _LANES - T), jnp.inf, jnp.float32)
    return jnp.concatenate([thresholds, pad, tr], axis=1)


@functools.partial(jax.jit, static_argnames=("cols_per_w", "chunk"))
def _run(xt, consts, cols_per_w, chunk):
    F, B = xt.shape
    n_chunks = cols_per_w // chunk
    w_per_f = B // cols_per_w
    mesh = plsc.VectorSubcoreMesh(core_axis_name="c", subcore_axis_name="s")

    @functools.partial(
        pl.kernel,
        mesh=mesh,
        out_type=jax.ShapeDtypeStruct((F, B), jnp.float32),
        compiler_params=pltpu.CompilerParams(needs_layout_passes=False),
        scratch_types=[
            pltpu.VMEM((_LANES,), jnp.float32),
            pltpu.VMEM((_LANES,), jnp.float32),
            pltpu.VMEM((chunk,), jnp.float32),
            pltpu.VMEM((chunk,), jnp.float32),
            pltpu.VMEM((chunk,), jnp.float32),
            pltpu.VMEM((chunk,), jnp.float32),
            pltpu.SemaphoreType.DMA,
            pltpu.SemaphoreType.DMA,
            pltpu.SemaphoreType.DMA,
            pltpu.SemaphoreType.DMA,
        ],
    )
    def run(x_hbm, c_hbm, out_hbm, th_v, tr_v,
            xb0, xb1, ob0, ob1, si0, si1, so0, so1):
        wid = lax.axis_index("s") * 2 + lax.axis_index("c")
        f = wid // w_per_f
        col0 = (wid % w_per_f) * cols_per_w
        pltpu.sync_copy(c_hbm.at[f, pl.ds(0, _LANES)], th_v)
        pltpu.sync_copy(c_hbm.at[f, pl.ds(_LANES, _LANES)], tr_v)
        th7 = th_v[...][7]

        xbufs, obufs = (xb0, xb1), (ob0, ob1)
        isems, osems = (si0, si1), (so0, so1)
        in_h = [None] * n_chunks
        out_h = [None] * n_chunks

        in_h[0] = pltpu.async_copy(
            x_hbm.at[f, pl.ds(col0, chunk)], xbufs[0], isems[0])
        for ci in range(n_chunks):
            cur = ci & 1
            if ci + 1 < n_chunks:
                nxt = col0 + (ci + 1) * chunk
                in_h[ci + 1] = pltpu.async_copy(
                    x_hbm.at[f, pl.ds(nxt, chunk)], xbufs[1 - cur],
                    isems[1 - cur])
            in_h[ci].wait()
            if ci >= 2:
                out_h[ci - 2].wait()
            xbuf, obuf = xbufs[cur], obufs[cur]

            @plsc.parallel_loop(0, chunk // _LANES, 1, unroll=8)
            def vec_body(r):
                xv = xbuf[pl.ds(r * _LANES, _LANES)]
                enc = jnp.where(xv > th7, jnp.int32(8), jnp.int32(0))
                pv = plsc.load_gather(th_v, [enc + 3])
                enc = jnp.where(xv > pv, enc + 4, enc)
                pv = plsc.load_gather(th_v, [enc + 1])
                enc = jnp.where(xv > pv, enc + 2, enc)
                pv = plsc.load_gather(th_v, [enc])
                enc = jnp.where(xv > pv, enc + 1, enc)
                obuf[pl.ds(r * _LANES, _LANES)] = plsc.load_gather(tr_v, [enc])

            out_h[ci] = pltpu.async_copy(
                obuf, out_hbm.at[f, pl.ds(col0 + ci * chunk, chunk)],
                osems[cur])
        for ci in (n_chunks - 2, n_chunks - 1):
            if ci >= 0:
                out_h[ci].wait()

    return run(xt, consts)


def kernel(x, thresholds):
    B, F = x.shape
    consts = _build_consts(thresholds)
    info = plsc.get_sparse_core_info()
    n_workers = info.num_cores * info.num_subcores
    cols_per_w = B // (n_workers // F)
    chunk = 8192
    out_t = _run(x.T, consts, cols_per_w, chunk)
    return out_t.T
